# Initial kernel scaffold; baseline (speedup 1.0000x reference)
import jax
import jax.numpy as jnp
from jax.experimental import pallas as pl


def kernel(f0, phone_label, phone_duration, midi_label, unvoiced_flag, W_f0, b_f0, phone_table, midi_table, W_unv, b_unv):
    raise NotImplementedError("write your pallas kernel here")


# speedup vs baseline: 2.1033x; 2.1033x over previous
import functools

import jax
import jax.numpy as jnp
from jax import lax
from jax.experimental import pallas as pl

BLK = 2048


def _body(f0_ref, ph_ref, md_ref, un_ref, pt_ref, mt_ref, wf_ref, bf_ref,
          wu_ref, bu_ref, of_ref, op_ref, om_ref, ou_ref):
    f0 = f0_ref[...]
    of_ref[...] = f0 * wf_ref[...] + bf_ref[...]
    un = un_ref[...]
    ou_ref[...] = un * wu_ref[...] + bu_ref[...]

    iota = lax.broadcasted_iota(jnp.int32, (BLK, 128), 1)
    oh_p = (ph_ref[...] == iota).astype(jnp.float32)
    op_ref[...] = jnp.dot(oh_p, pt_ref[...], preferred_element_type=jnp.float32)
    oh_m = (md_ref[...] == iota).astype(jnp.float32)
    om_ref[...] = jnp.dot(oh_m, mt_ref[...], preferred_element_type=jnp.float32)


@jax.jit
def _encode(f0r, phr, mdr, unr, pt_pad, mt, wf, bf, wu, bu):
    E = f0r.shape[0]
    grid = E // BLK
    blk_in = pl.BlockSpec((BLK, 1), lambda i: (i, 0))
    full = lambda shape: pl.BlockSpec(shape, lambda i: (0, 0))
    out_spec = lambda d: pl.BlockSpec((BLK, d), lambda i: (i, 0))
    return pl.pallas_call(
        _body,
        grid=(grid,),
        in_specs=[blk_in, blk_in, blk_in, blk_in,
                  full((128, 128)), full((128, 64)),
                  full((1, 64)), full((1, 64)),
                  full((1, 16)), full((1, 16))],
        out_specs=[out_spec(64), out_spec(128), out_spec(64), out_spec(16)],
        out_shape=[jax.ShapeDtypeStruct((E, 64), jnp.float32),
                   jax.ShapeDtypeStruct((E, 128), jnp.float32),
                   jax.ShapeDtypeStruct((E, 64), jnp.float32),
                   jax.ShapeDtypeStruct((E, 16), jnp.float32)],
    )(f0r, phr, mdr, unr, pt_pad, mt, wf, bf, wu, bu)


def kernel(f0, phone_label, phone_duration, midi_label, unvoiced_flag,
           W_f0, b_f0, phone_table, midi_table, W_unv, b_unv):
    B, S = phone_label.shape
    E = B * S
    f0r = f0.reshape(E, 1)
    phr = phone_label.astype(jnp.int32).reshape(E, 1)
    mdr = midi_label.astype(jnp.int32).reshape(E, 1)
    unr = unvoiced_flag.reshape(E, 1)
    V = phone_table.shape[0]
    pt_pad = jnp.pad(phone_table, ((0, 128 - V), (0, 0)))
    wf = W_f0.reshape(1, -1)
    bf = b_f0.reshape(1, -1)
    wu = W_unv.reshape(1, -1)
    bu = b_unv.reshape(1, -1)
    of, op, om, ou = _encode(f0r, phr, mdr, unr, pt_pad, midi_table, wf, bf, wu, bu)
    return (of.reshape(B, S, 64), op.reshape(B, S, 128),
            om.reshape(B, S, 64), ou.reshape(B, S, 16))
